# Initial kernel scaffold; baseline (speedup 1.0000x reference)
import jax
import jax.numpy as jnp
from jax.experimental import pallas as pl


def kernel(x, edge_index, Wl1, Wr1, b1, Wl2, Wr2, b2, Wc, bc):
    raise NotImplementedError("write your pallas kernel here")


# speedup vs baseline: 7.2998x; 7.2998x over previous
import functools

import jax
import jax.numpy as jnp
from jax import lax
from jax.experimental import pallas as pl
from jax.experimental.pallas import tpu as pltpu
from jax.experimental.pallas import tpu_sc as plsc

N_NODES = 10000
N_EDGES = 160000
D_IN = 256
D_HID = 512
N_CLASSES = 40
PC = 48
DHALF = 128
XW = DHALF + 16

NCORES = 2
NSUB = 16
N_PAD = 10240
STRIPE = N_PAD // NSUB

E_PER_SUB1 = N_EDGES // NSUB
CH1 = 80
NCH1 = E_PER_SUB1 // CH1

E_PER_SUB2 = N_EDGES // (2 * NSUB)
CH2 = 40
NCH2 = E_PER_SUB2 // CH2

RB = 2000
_F32 = jnp.float32


def _sage_sc_mesh():
    return plsc.VectorSubcoreMesh(core_axis_name="c", subcore_axis_name="s")


@functools.partial(
    pl.kernel,
    out_type=jax.ShapeDtypeStruct((2, N_PAD, XW), _F32),
    mesh=_sage_sc_mesh(),
    compiler_params=pltpu.CompilerParams(use_tc_tiling_on_sc=False),
    scratch_types=[
        pltpu.VMEM_SHARED((N_PAD, XW), _F32),
        pltpu.VMEM((NCH1, CH1), jnp.int32),
        pltpu.VMEM((NCH1, CH1), jnp.int32),
        pltpu.VMEM((CH1, XW), _F32),
        pltpu.SemaphoreType.DMA,
    ],
)
def _k1_aggregate(xs2, srcr, dstr, zrow,
                  agg_out,
                  acc_sh, src_v, dst_v, rows_v, sem):
    cid = lax.axis_index("c")
    sid = lax.axis_index("s")
    r0 = sid * STRIPE

    pltpu.sync_copy(zrow, acc_sh.at[pl.ds(r0, STRIPE)])

    pltpu.sync_copy(srcr.at[sid], src_v)
    pltpu.sync_copy(dstr.at[sid], dst_v)

    @pl.when(cid == 1)
    def _():
        def adj(i, _):
            r = i // (CH1 // 16)
            c = i % (CH1 // 16)
            src_v[r, pl.ds(c * 16, 16)] = src_v[r, pl.ds(c * 16, 16)] + N_NODES
            return 0
        lax.fori_loop(0, NCH1 * (CH1 // 16), adj, 0)

    plsc.subcore_barrier()

    def chunk(j, _):
        pltpu.async_copy(xs2.at[src_v.at[j]], rows_v, sem).wait()
        pltpu.sync_copy(rows_v, acc_sh.at[dst_v.at[j]], add=True)
        return 0

    lax.fori_loop(0, NCH1, chunk, 0)
    plsc.subcore_barrier()

    pltpu.sync_copy(acc_sh.at[pl.ds(r0, STRIPE)],
                    agg_out.at[cid, pl.ds(r0, STRIPE)])


@functools.partial(
    pl.kernel,
    out_type=jax.ShapeDtypeStruct((2, N_PAD, PC), _F32),
    mesh=_sage_sc_mesh(),
    compiler_params=pltpu.CompilerParams(use_tc_tiling_on_sc=False),
    scratch_types=[
        pltpu.VMEM_SHARED((N_PAD, PC), _F32),
        pltpu.VMEM((NCH2, CH2), jnp.int32),
        pltpu.VMEM((NCH2, CH2), jnp.int32),
        pltpu.VMEM((CH2, PC), _F32),
        pltpu.SemaphoreType.DMA,
    ],
)
def _k3_aggregate(p_hbm, srcr, dstr, zrow,
                  agg_out,
                  acc_sh, src_v, dst_v, rows_v, sem):
    cid = lax.axis_index("c")
    sid = lax.axis_index("s")
    r0 = sid * STRIPE

    pltpu.sync_copy(zrow, acc_sh.at[pl.ds(r0, STRIPE)])
    pltpu.sync_copy(srcr.at[cid, sid], src_v)
    pltpu.sync_copy(dstr.at[cid, sid], dst_v)
    plsc.subcore_barrier()

    def chunk(j, _):
        pltpu.async_copy(p_hbm.at[src_v.at[j]], rows_v, sem).wait()
        pltpu.sync_copy(rows_v, acc_sh.at[dst_v.at[j]], add=True)
        return 0

    lax.fori_loop(0, NCH2, chunk, 0)
    plsc.subcore_barrier()

    pltpu.sync_copy(acc_sh.at[pl.ds(r0, STRIPE)],
                    agg_out.at[cid, pl.ds(r0, STRIPE)])


def _k0_body(wl2, wr2, b2r, wcp, bcp, wlc_o, wrc_o, bcc_o):
    wlc_o[...] = jnp.dot(wl2[...], wcp[...], preferred_element_type=_F32)
    wrc_o[...] = jnp.dot(wr2[...], wcp[...], preferred_element_type=_F32)
    bcc_o[...] = jnp.dot(b2r[...], wcp[...], preferred_element_type=_F32) + bcp[...]


def _weight_fold(Wl2, Wr2, b2r, Wcp, bcp):
    return pl.pallas_call(
        _k0_body,
        out_shape=[
            jax.ShapeDtypeStruct((D_HID, PC), _F32),
            jax.ShapeDtypeStruct((D_HID, PC), _F32),
            jax.ShapeDtypeStruct((1, PC), _F32),
        ],
    )(Wl2, Wr2, b2r, Wcp, bcp)


def _k2_body(agg, x, wl1, wr1, b1, wlc, wrc, bcc, p_o, q_o):
    inv = 1.0 / jnp.maximum(agg[0][:, DHALF:DHALF + 1], 1.0)
    mlo = agg[0][:, 0:DHALF] * inv
    mhi = agg[1][:, 0:DHALF] * inv
    h = (jnp.dot(mlo, wl1[0:DHALF, :], preferred_element_type=_F32)
         + jnp.dot(mhi, wl1[DHALF:D_IN, :], preferred_element_type=_F32)
         + jnp.dot(x[...], wr1[...], preferred_element_type=_F32)
         + b1[...])
    h = jnp.maximum(h, 0.0)
    p_o[...] = jnp.dot(h, wlc[...], preferred_element_type=_F32)
    q_o[...] = jnp.dot(h, wrc[...], preferred_element_type=_F32) + bcc[...]


def _layer1_tc(agg1, x, Wl1, Wr1, b1r, WlC, WrC, bcc):
    nblk = N_NODES // RB
    full = lambda i: (0, 0)
    return pl.pallas_call(
        _k2_body,
        grid=(nblk,),
        in_specs=[
            pl.BlockSpec((2, RB, XW), lambda i: (0, i, 0)),
            pl.BlockSpec((RB, D_IN), lambda i: (i, 0)),
            pl.BlockSpec((D_IN, D_HID), full),
            pl.BlockSpec((D_IN, D_HID), full),
            pl.BlockSpec((1, D_HID), full),
            pl.BlockSpec((D_HID, PC), full),
            pl.BlockSpec((D_HID, PC), full),
            pl.BlockSpec((1, PC), full),
        ],
        out_specs=[
            pl.BlockSpec((RB, PC), lambda i: (i, 0)),
            pl.BlockSpec((RB, PC), lambda i: (i, 0)),
        ],
        out_shape=[
            jax.ShapeDtypeStruct((N_NODES, PC), _F32),
            jax.ShapeDtypeStruct((N_NODES, PC), _F32),
        ],
    )(agg1, x, Wl1, Wr1, b1r, WlC, WrC, bcc)


def _k4_body(agg2, agg1, q, out):
    inv = 1.0 / jnp.maximum(agg1[0][:, DHALF:DHALF + 1], 1.0)
    o = (agg2[0] + agg2[1]) * inv + q[...]
    out[...] = o[:, 0:N_CLASSES]


def _finalize_tc(agg2, agg1, q):
    nblk = N_NODES // RB
    return pl.pallas_call(
        _k4_body,
        grid=(nblk,),
        in_specs=[
            pl.BlockSpec((2, RB, PC), lambda i: (0, i, 0)),
            pl.BlockSpec((1, RB, XW), lambda i: (0, i, 0)),
            pl.BlockSpec((RB, PC), lambda i: (i, 0)),
        ],
        out_specs=pl.BlockSpec((RB, N_CLASSES), lambda i: (i, 0)),
        out_shape=jax.ShapeDtypeStruct((N_NODES, N_CLASSES), _F32),
    )(agg2, agg1, q)


def kernel(x, edge_index, Wl1, Wr1, b1, Wl2, Wr2, b2, Wc, bc):
    src = edge_index[0].astype(jnp.int32)
    dst = edge_index[1].astype(jnp.int32)

    ones_blk = jnp.ones((N_NODES, 16), _F32)
    xs2 = jnp.concatenate(
        [jnp.concatenate([x[:, :DHALF], ones_blk], axis=1),
         jnp.concatenate([x[:, DHALF:], ones_blk], axis=1)], axis=0)

    src1 = src.reshape(NSUB, NCH1, CH1)
    dst1 = dst.reshape(NSUB, NCH1, CH1)
    src2 = src.reshape(2, NSUB, NCH2, CH2)
    dst2 = dst.reshape(2, NSUB, NCH2, CH2)

    zx = jnp.zeros((STRIPE, XW), _F32)
    zp = jnp.zeros((STRIPE, PC), _F32)

    b1r = b1.reshape(1, D_HID)
    b2r = b2.reshape(1, D_HID)
    Wcp = jnp.pad(Wc, ((0, 0), (0, PC - N_CLASSES)))
    bcp = jnp.pad(bc, (0, PC - N_CLASSES)).reshape(1, PC)

    agg1 = _k1_aggregate(xs2, src1, dst1, zx)
    WlC, WrC, bcc = _weight_fold(Wl2, Wr2, b2r, Wcp, bcp)
    p, q = _layer1_tc(agg1, x, Wl1, Wr1, b1r, WlC, WrC, bcc)
    agg2 = _k3_aggregate(p, src2, dst2, zp)
    return _finalize_tc(agg2, agg1, q)
